# baseline (speedup 1.0000x reference)
import functools

import jax
import jax.numpy as jnp
from jax import lax
from jax.experimental import pallas as pl
from jax.experimental.pallas import tpu as pltpu
from jax.experimental.pallas import tpu_sc as plsc

_B, _P, _T, _D, _H = 4096, 8, 15, 256, 4
_DH = _D // _H
_UB = 64
_NG = _UB // 8
_R = 8 * _T
_NW = 32
_UPW = _B // _NW


_NCH = _UPW * _P // 128


def _sc_scatter(gnn_flat, ids_flat):
    mesh = plsc.VectorSubcoreMesh(core_axis_name="c", subcore_axis_name="s")

    @functools.partial(
        pl.kernel,
        out_type=jax.ShapeDtypeStruct((_B * _T + 128, _D), jnp.float32),
        mesh=mesh,
        scratch_types=[
            pltpu.VMEM((_UPW * _P + 16,), jnp.int32),
            pltpu.VMEM((_NCH, 128), jnp.int32),
            pltpu.VMEM((2, 128, _D), jnp.float32),
            pltpu.SemaphoreType.DMA,
            pltpu.SemaphoreType.DMA,
            pltpu.SemaphoreType.DMA,
        ],
    )
    def scat(gnn_hbm, ids_hbm, out_hbm, idsv, dstv, rows, gsem0, gsem1, ssem):
        gsems = [gsem0, gsem1]
        wid = lax.axis_index("s") * 2 + lax.axis_index("c")
        ebase = wid * _UPW * _P
        pltpu.sync_copy(ids_hbm.at[pl.ds(ebase, _UPW * _P)],
                        idsv.at[pl.ds(0, _UPW * _P)])
        gathers = []
        for j in range(2):
            gathers.append(pltpu.async_copy(
                gnn_hbm.at[pl.ds(ebase + j * 128, 128)], rows.at[j], gsems[j]))
        for k in range(_UPW * _P // 16):
            lane = lax.iota(jnp.int32, 16)
            e = ebase + k * 16 + lane
            u = lax.shift_right_logical(e, 3)
            v0 = idsv[pl.ds(k * 16, 16)]
            dst = u * _T + v0
            trash = _B * _T + (k % 8) * 16 + lane
            p_lane = lane & 7
            dup = None
            for s in range(1, _P):
                vs = idsv[pl.ds(k * 16 + s, 16)]
                vs = vs + lax.shift_right_logical(p_lane + s, 3) * 16
                same = v0 == vs
                dup = same if dup is None else dup | same
            dstv[k // 8, pl.ds((k % 8) * 16, 16)] = jnp.where(dup, trash, dst)
        for j in range(_NCH):
            b = j % 2
            gathers.pop(0).wait()
            cp = pltpu.async_copy(rows.at[b], out_hbm.at[dstv.at[j]], ssem)
            cp.wait()
            if j + 2 < _NCH:
                gathers.append(pltpu.async_copy(
                    gnn_hbm.at[pl.ds(ebase + (j + 2) * 128, 128)], rows.at[b],
                    gsems[b]))

    return scat(gnn_flat, ids_flat)


def _tc_body(buf_ref, ids2_ref, pos_ref, wq_ref, wk_ref, wv_ref,
             wo_ref, bq_ref, bk_ref, bv_ref, bo_ref, gam_ref, bet_ref,
             logits_ref, last_ref, am_ref, umax_ref):
    f32 = jnp.float32
    ids2 = ids2_ref[...]
    umax = jnp.max(ids2, axis=1, keepdims=True)
    umax_ref[...] = umax
    pos_tile = pos_ref[...]
    ids2f = ids2.astype(f32)
    ones_r = jnp.ones((_R, 1), f32)

    ri = lax.broadcasted_iota(jnp.int32, (_R, _R), 0) // _T
    ci = lax.broadcasted_iota(jnp.int32, (_R, _R), 1) // _T
    bd = jnp.where(ri == ci, 0.0, -1e9).astype(f32)

    r64 = lax.broadcasted_iota(jnp.int32, (_R, 64), 0)
    c64 = lax.broadcasted_iota(jnp.int32, (_R, 64), 1)
    u_r = r64 // _T
    t_r = (r64 - u_r * _T).astype(f32)
    same_u = u_r == (c64 >> 3)

    c512 = lax.broadcasted_iota(jnp.int32, (_UB * _P, _P), 0)
    p512 = lax.broadcasted_iota(jnp.int32, (_UB * _P, _P), 1)
    psel = jnp.where((c512 & 7) == p512, 1.0, 0.0)
    mm = lax.dot_general(psel, ids2f, (((1,), (1,)), ((), ())))
    cu = lax.broadcasted_iota(jnp.int32, (_UB * _P, _UB), 0) >> 3
    uu = lax.broadcasted_iota(jnp.int32, (_UB * _P, _UB), 1)
    idscol = jnp.sum(jnp.where(cu == uu, mm, 0.0), axis=1,
                     keepdims=True)

    xs, annegs, mbias = [], [], []
    for g in range(_NG):
        idsg = lax.slice(idscol, (g * 64, 0), (g * 64 + 64, 1))
        ids_b = lax.dot_general(ones_r, idsg,
                                (((1,), (1,)), ((), ())))
        hit = jnp.where(same_u & (ids_b == t_r), 1.0, 0.0)
        act = jnp.max(hit, axis=1, keepdims=True)
        anneg = (act - 1.0) * 10000.0
        annegs.append(anneg)
        mbias.append(lax.dot_general(ones_r, anneg,
                                     (((1,), (1,)), ((), ()))) + bd)
        bufg = buf_ref[pl.ds(g * _R, _R), :]
        xs.append(jnp.where(act > 0.5, bufg, 0.0) + pos_tile)
    x = jnp.concatenate(xs, axis=0)
    am_ref[...] = jnp.concatenate(annegs, axis=0)

    ctx_heads = []
    for h in range(_H):
        q = jnp.dot(x, wq_ref[pl.ds(h * _D, _D), :],
                    preferred_element_type=f32) + bq_ref[h:h + 1, :]
        k = jnp.dot(x, wk_ref[pl.ds(h * _D, _D), :],
                    preferred_element_type=f32) + bk_ref[h:h + 1, :]
        v = jnp.dot(x, wv_ref[pl.ds(h * _D, _D), :],
                    preferred_element_type=f32) + bv_ref[h:h + 1, :]
        cgs = []
        for g in range(_NG):
            qg = lax.slice(q, (g * _R, 0), (g * _R + _R, _DH))
            kg = lax.slice(k, (g * _R, 0), (g * _R + _R, _DH))
            vg = lax.slice(v, (g * _R, 0), (g * _R + _R, _DH))
            s = lax.dot_general(qg, kg, (((1,), (1,)), ((), ())),
                                preferred_element_type=f32)
            s = s + mbias[g]
            s = s - jnp.max(s, axis=1, keepdims=True)
            e = jnp.exp(s)
            probs = e / jnp.sum(e, axis=1, keepdims=True)
            cgs.append(jnp.dot(probs, vg, preferred_element_type=f32))
        ctx_heads.append(jnp.concatenate(cgs, axis=0))

    hres = bo_ref[...] + x
    for h in range(_H):
        hres = hres + jnp.dot(ctx_heads[h], wo_ref[pl.ds(h * _DH, _DH), :],
                              preferred_element_type=f32)
    mu = jnp.mean(hres, axis=1, keepdims=True)
    dev = hres - mu
    var = jnp.mean(dev * dev, axis=1, keepdims=True)
    logits = gam_ref[...] * (dev / jnp.sqrt(var + 1e-12)) + bet_ref[...]
    logits_ref[...] = logits

    umf = umax.astype(f32)
    j8 = lax.broadcasted_iota(jnp.int32, (8, _R), 1).astype(f32)
    u8 = lax.broadcasted_iota(jnp.int32, (8, _R), 0).astype(f32)
    lasts = []
    for g in range(_NG):
        umg = lax.slice(umf, (g * 8, 0), (g * 8 + 8, 1))
        um_b = lax.dot_general(umg, ones_r, (((1,), (1,)), ((), ())))
        one_hot = jnp.where(j8 == u8 * _T + um_b, 1.0, 0.0)
        lg = lax.slice(logits, (g * _R, 0), (g * _R + _R, _D))
        lasts.append(jnp.dot(one_hot, lg, preferred_element_type=f32))
    last_ref[...] = jnp.concatenate(lasts, axis=0)


def _tc_specs():
    grid = (_B // _UB,)
    rows = _UB * _T
    in_specs = [
        pl.BlockSpec((rows, _D), lambda i: (i, 0)),
        pl.BlockSpec((_UB, _P), lambda i: (i, 0)),
        pl.BlockSpec((_R, _D), lambda i: (0, 0)),
        pl.BlockSpec((_H * _D, _DH), lambda i: (0, 0)),
        pl.BlockSpec((_H * _D, _DH), lambda i: (0, 0)),
        pl.BlockSpec((_H * _D, _DH), lambda i: (0, 0)),
        pl.BlockSpec((_D, _D), lambda i: (0, 0)),
        pl.BlockSpec((_H, _DH), lambda i: (0, 0)),
        pl.BlockSpec((_H, _DH), lambda i: (0, 0)),
        pl.BlockSpec((_H, _DH), lambda i: (0, 0)),
        pl.BlockSpec((1, _D), lambda i: (0, 0)),
        pl.BlockSpec((1, _D), lambda i: (0, 0)),
        pl.BlockSpec((1, _D), lambda i: (0, 0)),
    ]
    out_specs = [
        pl.BlockSpec((rows, _D), lambda i: (i, 0)),
        pl.BlockSpec((_UB, _D), lambda i: (i, 0)),
        pl.BlockSpec((rows, 1), lambda i: (i, 0)),
        pl.BlockSpec((_UB, 1), lambda i: (i, 0)),
    ]
    out_shapes = [
        jax.ShapeDtypeStruct((_B * _T, _D), jnp.float32),
        jax.ShapeDtypeStruct((_B, _D), jnp.float32),
        jax.ShapeDtypeStruct((_B * _T, 1), jnp.float32),
        jax.ShapeDtypeStruct((_B, 1), jnp.int32),
    ]
    return grid, in_specs, out_specs, out_shapes


def _tc_attention(buf, period_ids, pos_emb, wq_r, wk_r, wv_r, Wo,
                  bq_r, bk_r, bv_r, bo_r, gam_r, bet_r):
    grid, in_specs, out_specs, out_shapes = _tc_specs()
    return pl.pallas_call(
        _tc_body,
        grid=grid,
        in_specs=in_specs,
        out_specs=out_specs,
        out_shape=out_shapes,
    )(buf, period_ids, pos_emb, wq_r, wk_r, wv_r, Wo,
      bq_r, bk_r, bv_r, bo_r, gam_r, bet_r)


def kernel(gnn_out, period_ids, pos_emb, Wq, bq, Wk, bk, Wv, bv, Wo, bo,
           ln_gamma, ln_beta):
    gnn_flat = gnn_out.reshape(_B * _P, _D)
    buf = _sc_scatter(gnn_flat, period_ids.reshape(-1))

    pos_tiled = jnp.tile(pos_emb, (8, 1))

    def per_head(w):
        return w.reshape(_D, _H, _DH).transpose(1, 0, 2).reshape(_H * _D, _DH)

    logits_f, last, am_f, umax_c = _tc_attention(
        buf, period_ids, pos_tiled,
        per_head(Wq) * 0.125, per_head(Wk), per_head(Wv), Wo,
        (bq * 0.125).reshape(_H, _DH), bk.reshape(_H, _DH),
        bv.reshape(_H, _DH),
        bo.reshape(1, _D), ln_gamma.reshape(1, _D), ln_beta.reshape(1, _D))

    logits = logits_f.reshape(_B, _T, _D)
    attention_mask = am_f.reshape(_B, _T)
    user_max_period = umax_c.reshape(_B)
    return logits, last, attention_mask, user_max_period


# speedup vs baseline: 1.0368x; 1.0368x over previous
import functools

import jax
import jax.numpy as jnp
from jax import lax
from jax.experimental import pallas as pl
from jax.experimental.pallas import tpu as pltpu
from jax.experimental.pallas import tpu_sc as plsc

_B, _P, _T, _D, _H = 4096, 8, 15, 256, 4
_DH = _D // _H
_UB = 64
_NG = _UB // 8
_R = 8 * _T
_NW = 32
_UPW = _B // _NW


_NCH = _UPW * _P // 128


def _sc_scatter(gnn_flat, ids_flat):
    mesh = plsc.VectorSubcoreMesh(core_axis_name="c", subcore_axis_name="s")

    @functools.partial(
        pl.kernel,
        out_type=jax.ShapeDtypeStruct((_B * _T + 128, _D), jnp.float32),
        mesh=mesh,
        scratch_types=[
            pltpu.VMEM((_UPW * _P + 16,), jnp.int32),
            pltpu.VMEM((_NCH, 128), jnp.int32),
            pltpu.VMEM((2, 128, _D), jnp.float32),
            pltpu.SemaphoreType.DMA,
            pltpu.SemaphoreType.DMA,
            pltpu.SemaphoreType.DMA,
        ],
    )
    def scat(gnn_hbm, ids_hbm, out_hbm, idsv, dstv, rows, gsem0, gsem1, ssem):
        gsems = [gsem0, gsem1]
        wid = lax.axis_index("s") * 2 + lax.axis_index("c")
        ebase = wid * _UPW * _P
        pltpu.sync_copy(ids_hbm.at[pl.ds(ebase, _UPW * _P)],
                        idsv.at[pl.ds(0, _UPW * _P)])
        gathers = []
        for j in range(2):
            gathers.append(pltpu.async_copy(
                gnn_hbm.at[pl.ds(ebase + j * 128, 128)], rows.at[j], gsems[j]))
        for k in range(_UPW * _P // 16):
            lane = lax.iota(jnp.int32, 16)
            e = ebase + k * 16 + lane
            u = lax.shift_right_logical(e, 3)
            v0 = idsv[pl.ds(k * 16, 16)]
            dst = u * _T + v0
            trash = _B * _T + (k % 8) * 16 + lane
            p_lane = lane & 7
            dup = None
            for s in range(1, _P):
                vs = idsv[pl.ds(k * 16 + s, 16)]
                vs = vs + lax.shift_right_logical(p_lane + s, 3) * 16
                same = v0 == vs
                dup = same if dup is None else dup | same
            dstv[k // 8, pl.ds((k % 8) * 16, 16)] = jnp.where(dup, trash, dst)
        for j in range(_NCH):
            b = j % 2
            gathers.pop(0).wait()
            cp = pltpu.async_copy(rows.at[b], out_hbm.at[dstv.at[j]], ssem)
            cp.wait()
            if j + 2 < _NCH:
                gathers.append(pltpu.async_copy(
                    gnn_hbm.at[pl.ds(ebase + (j + 2) * 128, 128)], rows.at[b],
                    gsems[b]))

    return scat(gnn_flat, ids_flat)


def _tc_body(buf_ref, idsc_ref, ids2_ref, pos_ref, wq_ref, wk_ref, wv_ref,
             wo_ref, bq_ref, bk_ref, bv_ref, bo_ref, gam_ref, bet_ref,
             logits_ref, last_ref, am_ref, umax_ref):
    f32 = jnp.float32
    ids2 = ids2_ref[...]
    umax = jnp.max(ids2, axis=1, keepdims=True)
    umax_ref[...] = umax
    pos_tile = pos_ref[...]
    idscol = idsc_ref[...].astype(f32)
    ones_r = jnp.ones((_R, 1), f32)

    ri = lax.broadcasted_iota(jnp.int32, (_R, _R), 0) // _T
    ci = lax.broadcasted_iota(jnp.int32, (_R, _R), 1) // _T
    bd = jnp.where(ri == ci, 0.0, -1e9).astype(f32)

    r64 = lax.broadcasted_iota(jnp.int32, (_R, 64), 0)
    c64 = lax.broadcasted_iota(jnp.int32, (_R, 64), 1)
    u_r = r64 // _T
    t_r = (r64 - u_r * _T).astype(f32)
    same_u = u_r == (c64 >> 3)

    xs, annegs, mbias = [], [], []
    for g in range(_NG):
        idsg = lax.slice(idscol, (g * 64, 0), (g * 64 + 64, 1))
        ids_b = lax.dot_general(ones_r, idsg,
                                (((1,), (1,)), ((), ())))
        hit = jnp.where(same_u & (ids_b == t_r), 1.0, 0.0)
        act = jnp.max(hit, axis=1, keepdims=True)
        anneg = (act - 1.0) * 10000.0
        annegs.append(anneg)
        mbias.append(lax.dot_general(ones_r, anneg,
                                     (((1,), (1,)), ((), ()))) + bd)
        bufg = buf_ref[pl.ds(g * _R, _R), :]
        xs.append(jnp.where(act > 0.5, bufg, 0.0) + pos_tile)
    x = jnp.concatenate(xs, axis=0)
    am_ref[...] = jnp.concatenate(annegs, axis=0)

    ctx_heads = []
    for h in range(_H):
        q = jnp.dot(x, wq_ref[pl.ds(h * _D, _D), :],
                    preferred_element_type=f32) + bq_ref[h:h + 1, :]
        k = jnp.dot(x, wk_ref[pl.ds(h * _D, _D), :],
                    preferred_element_type=f32) + bk_ref[h:h + 1, :]
        v = jnp.dot(x, wv_ref[pl.ds(h * _D, _D), :],
                    preferred_element_type=f32) + bv_ref[h:h + 1, :]
        cgs = []
        for g in range(_NG):
            qg = lax.slice(q, (g * _R, 0), (g * _R + _R, _DH))
            kg = lax.slice(k, (g * _R, 0), (g * _R + _R, _DH))
            vg = lax.slice(v, (g * _R, 0), (g * _R + _R, _DH))
            s = lax.dot_general(qg, kg, (((1,), (1,)), ((), ())),
                                preferred_element_type=f32)
            s = s + mbias[g]
            s = s - jnp.max(s, axis=1, keepdims=True)
            e = jnp.exp(s)
            probs = e / jnp.sum(e, axis=1, keepdims=True)
            cgs.append(jnp.dot(probs, vg, preferred_element_type=f32))
        ctx_heads.append(jnp.concatenate(cgs, axis=0))

    hres = bo_ref[...] + x
    for h in range(_H):
        hres = hres + jnp.dot(ctx_heads[h], wo_ref[pl.ds(h * _DH, _DH), :],
                              preferred_element_type=f32)
    mu = jnp.mean(hres, axis=1, keepdims=True)
    dev = hres - mu
    var = jnp.mean(dev * dev, axis=1, keepdims=True)
    logits = gam_ref[...] * (dev / jnp.sqrt(var + 1e-12)) + bet_ref[...]
    logits_ref[...] = logits

    umf = umax.astype(f32)
    j8 = lax.broadcasted_iota(jnp.int32, (8, _R), 1).astype(f32)
    u8 = lax.broadcasted_iota(jnp.int32, (8, _R), 0).astype(f32)
    lasts = []
    for g in range(_NG):
        umg = lax.slice(umf, (g * 8, 0), (g * 8 + 8, 1))
        um_b = lax.dot_general(umg, ones_r, (((1,), (1,)), ((), ())))
        one_hot = jnp.where(j8 == u8 * _T + um_b, 1.0, 0.0)
        lg = lax.slice(logits, (g * _R, 0), (g * _R + _R, _D))
        lasts.append(jnp.dot(one_hot, lg, preferred_element_type=f32))
    last_ref[...] = jnp.concatenate(lasts, axis=0)


def _tc_specs():
    grid = (_B // _UB,)
    rows = _UB * _T
    in_specs = [
        pl.BlockSpec((rows, _D), lambda i: (i, 0)),
        pl.BlockSpec((_UB * _P, 1), lambda i: (i, 0)),
        pl.BlockSpec((_UB, _P), lambda i: (i, 0)),
        pl.BlockSpec((_R, _D), lambda i: (0, 0)),
        pl.BlockSpec((_H * _D, _DH), lambda i: (0, 0)),
        pl.BlockSpec((_H * _D, _DH), lambda i: (0, 0)),
        pl.BlockSpec((_H * _D, _DH), lambda i: (0, 0)),
        pl.BlockSpec((_D, _D), lambda i: (0, 0)),
        pl.BlockSpec((_H, _DH), lambda i: (0, 0)),
        pl.BlockSpec((_H, _DH), lambda i: (0, 0)),
        pl.BlockSpec((_H, _DH), lambda i: (0, 0)),
        pl.BlockSpec((1, _D), lambda i: (0, 0)),
        pl.BlockSpec((1, _D), lambda i: (0, 0)),
        pl.BlockSpec((1, _D), lambda i: (0, 0)),
    ]
    out_specs = [
        pl.BlockSpec((rows, _D), lambda i: (i, 0)),
        pl.BlockSpec((_UB, _D), lambda i: (i, 0)),
        pl.BlockSpec((rows, 1), lambda i: (i, 0)),
        pl.BlockSpec((_UB, 1), lambda i: (i, 0)),
    ]
    out_shapes = [
        jax.ShapeDtypeStruct((_B * _T, _D), jnp.float32),
        jax.ShapeDtypeStruct((_B, _D), jnp.float32),
        jax.ShapeDtypeStruct((_B * _T, 1), jnp.float32),
        jax.ShapeDtypeStruct((_B, 1), jnp.int32),
    ]
    return grid, in_specs, out_specs, out_shapes


def _tc_attention(buf, ids_col, period_ids, pos_emb, wq_r, wk_r, wv_r, Wo,
                  bq_r, bk_r, bv_r, bo_r, gam_r, bet_r):
    grid, in_specs, out_specs, out_shapes = _tc_specs()
    return pl.pallas_call(
        _tc_body,
        grid=grid,
        in_specs=in_specs,
        out_specs=out_specs,
        out_shape=out_shapes,
    )(buf, ids_col, period_ids, pos_emb, wq_r, wk_r, wv_r, Wo,
      bq_r, bk_r, bv_r, bo_r, gam_r, bet_r)


def kernel(gnn_out, period_ids, pos_emb, Wq, bq, Wk, bk, Wv, bv, Wo, bo,
           ln_gamma, ln_beta):
    gnn_flat = gnn_out.reshape(_B * _P, _D)
    buf = _sc_scatter(gnn_flat, period_ids.reshape(-1))

    pos_tiled = jnp.tile(pos_emb, (8, 1))

    def per_head(w):
        return w.reshape(_D, _H, _DH).transpose(1, 0, 2).reshape(_H * _D, _DH)

    logits_f, last, am_f, umax_c = _tc_attention(
        buf, period_ids.reshape(_B * _P, 1), period_ids, pos_tiled,
        per_head(Wq) * 0.125, per_head(Wk), per_head(Wv), Wo,
        (bq * 0.125).reshape(_H, _DH), bk.reshape(_H, _DH),
        bv.reshape(_H, _DH),
        bo.reshape(1, _D), ln_gamma.reshape(1, _D), ln_beta.reshape(1, _D))

    logits = logits_f.reshape(_B, _T, _D)
    attention_mask = am_f.reshape(_B, _T)
    user_max_period = umax_c.reshape(_B)
    return logits, last, attention_mask, user_max_period


# speedup vs baseline: 1.0418x; 1.0048x over previous
import functools

import jax
import jax.numpy as jnp
from jax import lax
from jax.experimental import pallas as pl
from jax.experimental.pallas import tpu as pltpu
from jax.experimental.pallas import tpu_sc as plsc

_B, _P, _T, _D, _H = 4096, 8, 15, 256, 4
_DH = _D // _H
_UB = 64
_NG = _UB // 8
_R = 8 * _T
_NW = 32
_UPW = _B // _NW


_NCH = _UPW * _P // 128


def _sc_scatter(gnn_flat, ids_flat):
    mesh = plsc.VectorSubcoreMesh(core_axis_name="c", subcore_axis_name="s")

    @functools.partial(
        pl.kernel,
        out_type=jax.ShapeDtypeStruct((_B * _T + 128, _D), jnp.float32),
        mesh=mesh,
        scratch_types=[
            pltpu.VMEM((_UPW * _P + 16,), jnp.int32),
            pltpu.VMEM((_NCH, 128), jnp.int32),
            pltpu.VMEM((2, 128, _D), jnp.float32),
            pltpu.SemaphoreType.DMA,
            pltpu.SemaphoreType.DMA,
            pltpu.SemaphoreType.DMA,
        ],
    )
    def scat(gnn_hbm, ids_hbm, out_hbm, idsv, dstv, rows, gsem0, gsem1, ssem):
        gsems = [gsem0, gsem1]
        wid = lax.axis_index("s") * 2 + lax.axis_index("c")
        ebase = wid * _UPW * _P
        pltpu.sync_copy(ids_hbm.at[pl.ds(ebase, _UPW * _P)],
                        idsv.at[pl.ds(0, _UPW * _P)])
        gathers = []
        for j in range(2):
            gathers.append(pltpu.async_copy(
                gnn_hbm.at[pl.ds(ebase + j * 128, 128)], rows.at[j], gsems[j]))
        for k in range(_UPW * _P // 16):
            lane = lax.iota(jnp.int32, 16)
            e = ebase + k * 16 + lane
            u = lax.shift_right_logical(e, 3)
            v0 = idsv[pl.ds(k * 16, 16)]
            dst = u * _T + v0
            trash = _B * _T + (k % 8) * 16 + lane
            p_lane = lane & 7
            dup = None
            for s in range(1, _P):
                vs = idsv[pl.ds(k * 16 + s, 16)]
                vs = vs + lax.shift_right_logical(p_lane + s, 3) * 16
                same = v0 == vs
                dup = same if dup is None else dup | same
            dstv[k // 8, pl.ds((k % 8) * 16, 16)] = jnp.where(dup, trash, dst)
        for j in range(_NCH):
            b = j % 2
            gathers.pop(0).wait()
            cp = pltpu.async_copy(rows.at[b], out_hbm.at[dstv.at[j]], ssem)
            cp.wait()
            if j + 2 < _NCH:
                gathers.append(pltpu.async_copy(
                    gnn_hbm.at[pl.ds(ebase + (j + 2) * 128, 128)], rows.at[b],
                    gsems[b]))

    return scat(gnn_flat, ids_flat)


def _tc_body(buf_ref, idsc_ref, ids2_ref, pos_ref, wq_ref, wk_ref, wv_ref,
             wo_ref, bq_ref, bk_ref, bv_ref, bo_ref, gam_ref, bet_ref,
             logits_ref, last_ref, am_ref, umax_ref):
    f32 = jnp.float32
    ids2 = ids2_ref[...]
    umax = jnp.max(ids2, axis=1, keepdims=True)
    umax_ref[...] = umax
    pos_tile = pos_ref[...]
    idscol = idsc_ref[...].astype(f32)
    ones_r = jnp.ones((_R, 1), f32)

    ri = lax.broadcasted_iota(jnp.int32, (_R, _R), 0) // _T
    ci = lax.broadcasted_iota(jnp.int32, (_R, _R), 1) // _T
    bd = jnp.where(ri == ci, 0.0, -1e9).astype(f32)

    r64 = lax.broadcasted_iota(jnp.int32, (_R, 64), 0)
    c64 = lax.broadcasted_iota(jnp.int32, (_R, 64), 1)
    u_r = r64 // _T
    t_r = (r64 - u_r * _T).astype(f32)
    same_u = u_r == (c64 >> 3)

    t15 = lax.broadcasted_iota(jnp.int32, (_UB, _T), 1).astype(f32)
    hit15 = None
    for p in range(_P):
        idp = lax.slice(ids2, (0, p), (_UB, p + 1)).astype(f32)
        eq = jnp.where(idp == t15, 1.0, 0.0)
        hit15 = eq if hit15 is None else jnp.maximum(hit15, eq)
    am_ref[...] = (hit15 - 1.0) * 10000.0

    xs, annegs, mbias = [], [], []
    for g in range(_NG):
        idsg = lax.slice(idscol, (g * 64, 0), (g * 64 + 64, 1))
        ids_b = lax.dot_general(ones_r, idsg,
                                (((1,), (1,)), ((), ())))
        hit = jnp.where(same_u & (ids_b == t_r), 1.0, 0.0)
        act = jnp.max(hit, axis=1, keepdims=True)
        anneg = (act - 1.0) * 10000.0
        annegs.append(anneg)
        mbias.append(lax.dot_general(ones_r, anneg,
                                     (((1,), (1,)), ((), ()))) + bd)
        bufg = buf_ref[pl.ds(g * _R, _R), :]
        xs.append(jnp.where(act > 0.5, bufg, 0.0) + pos_tile)
    x = jnp.concatenate(xs, axis=0)

    ctx_heads = []
    for h in range(_H):
        q = jnp.dot(x, wq_ref[pl.ds(h * _D, _D), :],
                    preferred_element_type=f32) + bq_ref[h:h + 1, :]
        k = jnp.dot(x, wk_ref[pl.ds(h * _D, _D), :],
                    preferred_element_type=f32) + bk_ref[h:h + 1, :]
        v = jnp.dot(x, wv_ref[pl.ds(h * _D, _D), :],
                    preferred_element_type=f32) + bv_ref[h:h + 1, :]
        cgs = []
        for g in range(_NG):
            qg = lax.slice(q, (g * _R, 0), (g * _R + _R, _DH))
            kg = lax.slice(k, (g * _R, 0), (g * _R + _R, _DH))
            vg = lax.slice(v, (g * _R, 0), (g * _R + _R, _DH))
            s = lax.dot_general(qg, kg, (((1,), (1,)), ((), ())),
                                preferred_element_type=f32)
            s = s + mbias[g]
            s = s - jnp.max(s, axis=1, keepdims=True)
            e = jnp.exp(s)
            probs = e / jnp.sum(e, axis=1, keepdims=True)
            cgs.append(jnp.dot(probs, vg, preferred_element_type=f32))
        ctx_heads.append(jnp.concatenate(cgs, axis=0))

    hres = bo_ref[...] + x
    for h in range(_H):
        hres = hres + jnp.dot(ctx_heads[h], wo_ref[pl.ds(h * _DH, _DH), :],
                              preferred_element_type=f32)
    mu = jnp.mean(hres, axis=1, keepdims=True)
    dev = hres - mu
    var = jnp.mean(dev * dev, axis=1, keepdims=True)
    logits = gam_ref[...] * (dev / jnp.sqrt(var + 1e-12)) + bet_ref[...]
    logits_ref[...] = logits

    umf = umax.astype(f32)
    j8 = lax.broadcasted_iota(jnp.int32, (8, _R), 1).astype(f32)
    u8 = lax.broadcasted_iota(jnp.int32, (8, _R), 0).astype(f32)
    lasts = []
    for g in range(_NG):
        umg = lax.slice(umf, (g * 8, 0), (g * 8 + 8, 1))
        um_b = lax.dot_general(umg, ones_r, (((1,), (1,)), ((), ())))
        one_hot = jnp.where(j8 == u8 * _T + um_b, 1.0, 0.0)
        lg = lax.slice(logits, (g * _R, 0), (g * _R + _R, _D))
        lasts.append(jnp.dot(one_hot, lg, preferred_element_type=f32))
    last_ref[...] = jnp.concatenate(lasts, axis=0)


def _tc_specs():
    grid = (_B // _UB,)
    rows = _UB * _T
    in_specs = [
        pl.BlockSpec((rows, _D), lambda i: (i, 0)),
        pl.BlockSpec((_UB * _P, 1), lambda i: (i, 0)),
        pl.BlockSpec((_UB, _P), lambda i: (i, 0)),
        pl.BlockSpec((_R, _D), lambda i: (0, 0)),
        pl.BlockSpec((_H * _D, _DH), lambda i: (0, 0)),
        pl.BlockSpec((_H * _D, _DH), lambda i: (0, 0)),
        pl.BlockSpec((_H * _D, _DH), lambda i: (0, 0)),
        pl.BlockSpec((_D, _D), lambda i: (0, 0)),
        pl.BlockSpec((_H, _DH), lambda i: (0, 0)),
        pl.BlockSpec((_H, _DH), lambda i: (0, 0)),
        pl.BlockSpec((_H, _DH), lambda i: (0, 0)),
        pl.BlockSpec((1, _D), lambda i: (0, 0)),
        pl.BlockSpec((1, _D), lambda i: (0, 0)),
        pl.BlockSpec((1, _D), lambda i: (0, 0)),
    ]
    out_specs = [
        pl.BlockSpec((rows, _D), lambda i: (i, 0)),
        pl.BlockSpec((_UB, _D), lambda i: (i, 0)),
        pl.BlockSpec((_UB, _T), lambda i: (i, 0)),
        pl.BlockSpec((_UB, 1), lambda i: (i, 0)),
    ]
    out_shapes = [
        jax.ShapeDtypeStruct((_B * _T, _D), jnp.float32),
        jax.ShapeDtypeStruct((_B, _D), jnp.float32),
        jax.ShapeDtypeStruct((_B, _T), jnp.float32),
        jax.ShapeDtypeStruct((_B, 1), jnp.int32),
    ]
    return grid, in_specs, out_specs, out_shapes


def _tc_attention(buf, ids_col, period_ids, pos_emb, wq_r, wk_r, wv_r, Wo,
                  bq_r, bk_r, bv_r, bo_r, gam_r, bet_r):
    grid, in_specs, out_specs, out_shapes = _tc_specs()
    return pl.pallas_call(
        _tc_body,
        grid=grid,
        in_specs=in_specs,
        out_specs=out_specs,
        out_shape=out_shapes,
    )(buf, ids_col, period_ids, pos_emb, wq_r, wk_r, wv_r, Wo,
      bq_r, bk_r, bv_r, bo_r, gam_r, bet_r)


def kernel(gnn_out, period_ids, pos_emb, Wq, bq, Wk, bk, Wv, bv, Wo, bo,
           ln_gamma, ln_beta):
    gnn_flat = gnn_out.reshape(_B * _P, _D)
    buf = _sc_scatter(gnn_flat, period_ids.reshape(-1))

    pos_tiled = jnp.tile(pos_emb, (8, 1))

    def per_head(w):
        return w.reshape(_D, _H, _DH).transpose(1, 0, 2).reshape(_H * _D, _DH)

    logits_f, last, am_f, umax_c = _tc_attention(
        buf, period_ids.reshape(_B * _P, 1), period_ids, pos_tiled,
        per_head(Wq) * 0.125, per_head(Wk), per_head(Wv), Wo,
        (bq * 0.125).reshape(_H, _DH), bk.reshape(_H, _DH),
        bv.reshape(_H, _DH),
        bo.reshape(1, _D), ln_gamma.reshape(1, _D), ln_beta.reshape(1, _D))

    logits = logits_f.reshape(_B, _T, _D)
    user_max_period = umax_c.reshape(_B)
    return logits, last, am_f, user_max_period
